# Initial kernel scaffold; baseline (speedup 1.0000x reference)
import jax
import jax.numpy as jnp
from jax.experimental import pallas as pl


def kernel(form_vec, mlp_w1, mlp_b1, mlp_w2, mlp_b2, conv1_w, conv1_b, conv2_w, conv2_b, lin_w1, lin_b1, lin_w2, lin_b2):
    raise NotImplementedError("write your pallas kernel here")


# speedup vs baseline: 9.0555x; 9.0555x over previous
import jax
import jax.numpy as jnp
from jax.experimental import pallas as pl

B, N, D, H = 2048, 18, 21, 512
G = 128
C1 = float(1.0 / (18.0 ** 0.5))
C2 = float(1.0 / 18.0)


def _agg_relu(h, bias):
    hub = h[(N - 1) * G:, :]
    leaf_sum = h[:G, :]
    for n in range(1, N - 1):
        leaf_sum = leaf_sum + h[n * G:(n + 1) * G, :]
    hub_new = C1 * leaf_sum + C2 * hub
    out = jnp.concatenate([h[:(N - 1) * G, :], hub_new], axis=0)
    return jnp.maximum(out + bias, 0.0)


def _body(fv_ref, mw1_ref, mb1_ref, mw2_ref, mb2_ref, w1_ref, b1_ref,
          w2_ref, b2_ref, lw1_ref, lb1_ref, lw2_ref, lb2_ref, out_ref):
    fv = fv_ref[...].reshape(N * G, D)
    t = fv[:, 16:21]
    a = jnp.maximum(jnp.dot(t, mw1_ref[...]) + mb1_ref[...], 0.0)
    nodes = jnp.maximum(jnp.dot(a, mw2_ref[...]) + mb2_ref[...], 0.0)
    h = (jnp.dot(fv[:, :16], w1_ref[:16, :]) + nodes * w1_ref[16:17, :])
    h = _agg_relu(h, b1_ref[...])
    h = _agg_relu(jnp.dot(h, w2_ref[...]), b2_ref[...])
    h = jnp.maximum(jnp.dot(h, lw1_ref[...]) + lb1_ref[...], 0.0)
    h = jnp.maximum(jnp.dot(h, lw2_ref[...]) + lb2_ref[...], 0.0)
    out_ref[...] = h.reshape(N, G, H)


def kernel(form_vec, mlp_w1, mlp_b1, mlp_w2, mlp_b2, conv1_w, conv1_b,
           conv2_w, conv2_b, lin_w1, lin_b1, lin_w2, lin_b2):
    fvt = form_vec.transpose(1, 0, 2)
    rep2 = lambda a: (a.shape, lambda i: (0, 0))
    w_specs = [
        pl.BlockSpec(*rep2(mlp_w1)),
        pl.BlockSpec((1, 5), lambda i: (0, 0)),
        pl.BlockSpec(*rep2(mlp_w2)),
        pl.BlockSpec((1, 1), lambda i: (0, 0)),
        pl.BlockSpec(*rep2(conv1_w)),
        pl.BlockSpec((1, H), lambda i: (0, 0)),
        pl.BlockSpec(*rep2(conv2_w)),
        pl.BlockSpec((1, H), lambda i: (0, 0)),
        pl.BlockSpec(*rep2(lin_w1)),
        pl.BlockSpec((1, H), lambda i: (0, 0)),
        pl.BlockSpec(*rep2(lin_w2)),
        pl.BlockSpec((1, H), lambda i: (0, 0)),
    ]
    out = pl.pallas_call(
        _body,
        grid=(B // G,),
        in_specs=[pl.BlockSpec((N, G, D), lambda i: (0, i, 0))] + w_specs,
        out_specs=pl.BlockSpec((N, G, H), lambda i: (0, i, 0)),
        out_shape=jax.ShapeDtypeStruct((N, B, H), jnp.float32),
    )(fvt, mlp_w1, mlp_b1.reshape(1, 5), mlp_w2, mlp_b2.reshape(1, 1),
      conv1_w, conv1_b.reshape(1, H), conv2_w, conv2_b.reshape(1, H),
      lin_w1, lin_b1.reshape(1, H), lin_w2, lin_b2.reshape(1, H))
    return out.transpose(1, 0, 2).reshape(B * N, H)


# speedup vs baseline: 11.0742x; 1.2229x over previous
import jax
import jax.numpy as jnp
from jax.experimental import pallas as pl

B, N, D, H = 2048, 18, 21, 512
G = 128
C1 = float(1.0 / (18.0 ** 0.5))
C2 = float(1.0 / 18.0)


def _agg_relu(h, bias):
    hub = h[(N - 1) * G:, :]
    leaf_sum = h[:G, :]
    for n in range(1, N - 1):
        leaf_sum = leaf_sum + h[n * G:(n + 1) * G, :]
    hub_new = C1 * leaf_sum + C2 * hub
    out = jnp.concatenate([h[:(N - 1) * G, :], hub_new], axis=0)
    return jnp.maximum(out + bias, 0.0)


def _body(fv_ref, mw1_ref, mb1_ref, mw2_ref, mb2_ref, w1_ref, b1_ref,
          w2_ref, b2_ref, lw1_ref, lb1_ref, lw2_ref, lb2_ref, out_ref):
    fv = fv_ref[...].swapaxes(0, 1).reshape(N * G, D)
    t = fv[:, 16:21]
    a = jnp.maximum(jnp.dot(t, mw1_ref[...]) + mb1_ref[...], 0.0)
    nodes = jnp.maximum(jnp.dot(a, mw2_ref[...]) + mb2_ref[...], 0.0)
    h = (jnp.dot(fv[:, :16], w1_ref[:16, :]) + nodes * w1_ref[16:17, :])
    h = _agg_relu(h, b1_ref[...])
    h = _agg_relu(jnp.dot(h, w2_ref[...]), b2_ref[...])
    h = jnp.maximum(jnp.dot(h, lw1_ref[...]) + lb1_ref[...], 0.0)
    h = jnp.maximum(jnp.dot(h, lw2_ref[...]) + lb2_ref[...], 0.0)
    out_ref[...] = h.reshape(N, G, H).swapaxes(0, 1)


def kernel(form_vec, mlp_w1, mlp_b1, mlp_w2, mlp_b2, conv1_w, conv1_b,
           conv2_w, conv2_b, lin_w1, lin_b1, lin_w2, lin_b2):
    rep2 = lambda a: (a.shape, lambda i: (0, 0))
    w_specs = [
        pl.BlockSpec(*rep2(mlp_w1)),
        pl.BlockSpec((1, 5), lambda i: (0, 0)),
        pl.BlockSpec(*rep2(mlp_w2)),
        pl.BlockSpec((1, 1), lambda i: (0, 0)),
        pl.BlockSpec(*rep2(conv1_w)),
        pl.BlockSpec((1, H), lambda i: (0, 0)),
        pl.BlockSpec(*rep2(conv2_w)),
        pl.BlockSpec((1, H), lambda i: (0, 0)),
        pl.BlockSpec(*rep2(lin_w1)),
        pl.BlockSpec((1, H), lambda i: (0, 0)),
        pl.BlockSpec(*rep2(lin_w2)),
        pl.BlockSpec((1, H), lambda i: (0, 0)),
    ]
    out = pl.pallas_call(
        _body,
        grid=(B // G,),
        in_specs=[pl.BlockSpec((G, N, D), lambda i: (i, 0, 0))] + w_specs,
        out_specs=pl.BlockSpec((G, N, H), lambda i: (i, 0, 0)),
        out_shape=jax.ShapeDtypeStruct((B, N, H), jnp.float32),
    )(form_vec, mlp_w1, mlp_b1.reshape(1, 5), mlp_w2, mlp_b2.reshape(1, 1),
      conv1_w, conv1_b.reshape(1, H), conv2_w, conv2_b.reshape(1, H),
      lin_w1, lin_b1.reshape(1, H), lin_w2, lin_b2.reshape(1, H))
    return out.reshape(B * N, H)


# speedup vs baseline: 12.5813x; 1.1361x over previous
import jax
import jax.numpy as jnp
from jax.experimental import pallas as pl

B, N, D, H = 2048, 18, 21, 512
G = 128
C1 = float(1.0 / (18.0 ** 0.5))
C2 = float(1.0 / 18.0)


def _agg_relu(h, bias):
    hub = h[(N - 1) * G:, :]
    leaf_sum = h[:G, :]
    for n in range(1, N - 1):
        leaf_sum = leaf_sum + h[n * G:(n + 1) * G, :]
    hub_new = C1 * leaf_sum + C2 * hub
    out = jnp.concatenate([h[:(N - 1) * G, :], hub_new], axis=0)
    return jnp.maximum(out + bias, 0.0)


def _body(fv_ref, mw1_ref, mb1_ref, mw2_ref, mb2_ref, w1_ref, b1_ref,
          w2_ref, b2_ref, lw1_ref, lb1_ref, lw2_ref, lb2_ref, out_ref):
    fv = fv_ref[...].swapaxes(0, 1).reshape(N * G, D)
    t = fv[:, 16:21]
    a = jnp.maximum(jnp.dot(t, mw1_ref[...]) + mb1_ref[...], 0.0)
    nodes = jnp.maximum(jnp.dot(a, mw2_ref[...]) + mb2_ref[...], 0.0)
    h = (jnp.dot(fv[:, :16], w1_ref[:16, :]) + nodes * w1_ref[16:17, :])
    h = _agg_relu(h, b1_ref[...])
    h = _agg_relu(jnp.dot(h, w2_ref[...]), b2_ref[...])
    h = jnp.maximum(jnp.dot(h, lw1_ref[...]) + lb1_ref[...], 0.0)
    h = jnp.maximum(jnp.dot(h, lw2_ref[...]) + lb2_ref[...], 0.0)
    out_ref[...] = jnp.concatenate([h[n * G:(n + 1) * G, :] for n in range(N)], axis=1)


def kernel(form_vec, mlp_w1, mlp_b1, mlp_w2, mlp_b2, conv1_w, conv1_b,
           conv2_w, conv2_b, lin_w1, lin_b1, lin_w2, lin_b2):
    rep2 = lambda a: (a.shape, lambda i: (0, 0))
    w_specs = [
        pl.BlockSpec(*rep2(mlp_w1)),
        pl.BlockSpec((1, 5), lambda i: (0, 0)),
        pl.BlockSpec(*rep2(mlp_w2)),
        pl.BlockSpec((1, 1), lambda i: (0, 0)),
        pl.BlockSpec(*rep2(conv1_w)),
        pl.BlockSpec((1, H), lambda i: (0, 0)),
        pl.BlockSpec(*rep2(conv2_w)),
        pl.BlockSpec((1, H), lambda i: (0, 0)),
        pl.BlockSpec(*rep2(lin_w1)),
        pl.BlockSpec((1, H), lambda i: (0, 0)),
        pl.BlockSpec(*rep2(lin_w2)),
        pl.BlockSpec((1, H), lambda i: (0, 0)),
    ]
    out = pl.pallas_call(
        _body,
        grid=(B // G,),
        in_specs=[pl.BlockSpec((G, N, D), lambda i: (i, 0, 0))] + w_specs,
        out_specs=pl.BlockSpec((G, N * H), lambda i: (i, 0)),
        out_shape=jax.ShapeDtypeStruct((B, N * H), jnp.float32),
    )(form_vec, mlp_w1, mlp_b1.reshape(1, 5), mlp_w2, mlp_b2.reshape(1, 1),
      conv1_w, conv1_b.reshape(1, H), conv2_w, conv2_b.reshape(1, H),
      lin_w1, lin_b1.reshape(1, H), lin_w2, lin_b2.reshape(1, H))
    return out.reshape(B * N, H)


# speedup vs baseline: 18.0923x; 1.4380x over previous
import jax
import jax.numpy as jnp
from jax.experimental import pallas as pl

B, N, D, H = 2048, 18, 21, 512
G = 128
C1 = float(1.0 / (18.0 ** 0.5))
C2 = float(1.0 / 18.0)


def _agg_relu(h, bias):
    hub = h[(N - 1) * G:, :]
    leaf_sum = h[:G, :]
    for n in range(1, N - 1):
        leaf_sum = leaf_sum + h[n * G:(n + 1) * G, :]
    hub_new = C1 * leaf_sum + C2 * hub
    out = jnp.concatenate([h[:(N - 1) * G, :], hub_new], axis=0)
    return jnp.maximum(out + bias, 0.0)


def _body(fv_ref, mw1_ref, mb1_ref, mw2_ref, mb2_ref, w1_ref, b1_ref,
          w2_ref, b2_ref, lw1_ref, lb1_ref, lw2_ref, lb2_ref, out_ref):
    fv = fv_ref[...].swapaxes(0, 1).reshape(N * G, D)
    t = fv[:, 16:21]
    a = jnp.maximum(jnp.dot(t, mw1_ref[...]) + mb1_ref[...], 0.0)
    nodes = jnp.maximum(jnp.dot(a, mw2_ref[...]) + mb2_ref[...], 0.0)
    h = (jnp.dot(fv[:, :16], w1_ref[:16, :]) + nodes * w1_ref[16:17, :])
    h = _agg_relu(h, b1_ref[...])
    h = _agg_relu(jnp.dot(h, w2_ref[...]), b2_ref[...])
    h = jnp.maximum(jnp.dot(h, lw1_ref[...]) + lb1_ref[...], 0.0)
    h = jnp.maximum(jnp.dot(h, lw2_ref[...]) + lb2_ref[...], 0.0)
    out_ref[...] = h.reshape(N, G, H).swapaxes(0, 1).reshape(N * G, H)


def kernel(form_vec, mlp_w1, mlp_b1, mlp_w2, mlp_b2, conv1_w, conv1_b,
           conv2_w, conv2_b, lin_w1, lin_b1, lin_w2, lin_b2):
    rep2 = lambda a: (a.shape, lambda i: (0, 0))
    w_specs = [
        pl.BlockSpec(*rep2(mlp_w1)),
        pl.BlockSpec((1, 5), lambda i: (0, 0)),
        pl.BlockSpec(*rep2(mlp_w2)),
        pl.BlockSpec((1, 1), lambda i: (0, 0)),
        pl.BlockSpec(*rep2(conv1_w)),
        pl.BlockSpec((1, H), lambda i: (0, 0)),
        pl.BlockSpec(*rep2(conv2_w)),
        pl.BlockSpec((1, H), lambda i: (0, 0)),
        pl.BlockSpec(*rep2(lin_w1)),
        pl.BlockSpec((1, H), lambda i: (0, 0)),
        pl.BlockSpec(*rep2(lin_w2)),
        pl.BlockSpec((1, H), lambda i: (0, 0)),
    ]
    out = pl.pallas_call(
        _body,
        grid=(B // G,),
        in_specs=[pl.BlockSpec((G, N, D), lambda i: (i, 0, 0))] + w_specs,
        out_specs=pl.BlockSpec((G * N, H), lambda i: (i, 0)),
        out_shape=jax.ShapeDtypeStruct((B * N, H), jnp.float32),
    )(form_vec, mlp_w1, mlp_b1.reshape(1, 5), mlp_w2, mlp_b2.reshape(1, 1),
      conv1_w, conv1_b.reshape(1, H), conv2_w, conv2_b.reshape(1, H),
      lin_w1, lin_b1.reshape(1, H), lin_w2, lin_b2.reshape(1, H))
    return out.reshape(B * N, H)
